# Initial kernel scaffold; baseline (speedup 1.0000x reference)
import jax
import jax.numpy as jnp
from jax.experimental import pallas as pl


def kernel(x, edge_index, fc1_w0, fc1_b0, fc2_w0, fc2_b0, conv_b0, gamma0, beta0, fc1_w1, fc1_b1, fc2_w1, fc2_b1, conv_b1, gamma1, beta1, base_w, spline_w):
    raise NotImplementedError("write your pallas kernel here")


# speedup vs baseline: 10.1874x; 10.1874x over previous
import functools

import jax
import jax.numpy as jnp
from jax import lax
from jax.experimental import pallas as pl
from jax.experimental.pallas import tpu as pltpu
from jax.experimental.pallas import tpu_sc as plsc

N = 10000
E = 320000
D = 128
C_OUT = 40

NC = 2
NS = 16
NW = NC * NS
EPW = E // NW
CH = 80
NCH = EPW // CH
RPT = 624
TAIL = N - NS * RPT
DEGW = 128

BLK = 1000
NBLK = N // BLK

_sc_mesh = plsc.VectorSubcoreMesh(core_axis_name="c", subcore_axis_name="s")


@functools.partial(
    pl.kernel,
    mesh=_sc_mesh,
    out_type=jax.ShapeDtypeStruct((NC, N, DEGW), jnp.float32),
    scratch_types=[
        pltpu.VMEM((CH,), jnp.int32),
        pltpu.VMEM((CH, DEGW), jnp.float32),
        pltpu.VMEM_SHARED((N, DEGW), jnp.float32),
    ],
)
def _sc_degree(dst_hbm, ones_hbm, zeros_hbm, out_hbm, idx_v, ones_v, deg_sh):
    c = lax.axis_index("c")
    s = lax.axis_index("s")
    wid = c * NS + s
    pltpu.sync_copy(zeros_hbm, deg_sh.at[pl.ds(s * RPT, RPT)])

    @pl.when(s == 0)
    def _():
        pltpu.sync_copy(zeros_hbm.at[pl.ds(0, TAIL)],
                        deg_sh.at[pl.ds(NS * RPT, TAIL)])

    pltpu.sync_copy(ones_hbm, ones_v)
    plsc.subcore_barrier()
    base = wid * EPW

    def body(ci, carry):
        off = pl.multiple_of(base + ci * CH, 8)
        pltpu.sync_copy(dst_hbm.at[pl.ds(off, CH)], idx_v)
        pltpu.sync_copy(ones_v, deg_sh.at[idx_v], add=True)
        return carry

    lax.fori_loop(0, NCH, body, 0)
    plsc.subcore_barrier()
    pltpu.sync_copy(deg_sh.at[pl.ds(s * RPT, RPT)],
                    out_hbm.at[c, pl.ds(s * RPT, RPT)])

    @pl.when(s == 0)
    def _():
        pltpu.sync_copy(deg_sh.at[pl.ds(NS * RPT, TAIL)],
                        out_hbm.at[c, pl.ds(NS * RPT, TAIL)])


@functools.partial(
    pl.kernel,
    mesh=_sc_mesh,
    out_type=jax.ShapeDtypeStruct((NC, N, D), jnp.float32),
    scratch_types=[
        pltpu.VMEM((CH,), jnp.int32),
        pltpu.VMEM((CH,), jnp.int32),
        pltpu.VMEM((CH, D), jnp.float32),
        pltpu.VMEM_SHARED((N, D), jnp.float32),
        pltpu.SemaphoreType.DMA,
    ],
)
def _sc_aggregate(src_hbm, dst_hbm, hs_hbm, zeros_hbm, out_hbm,
                  src_v, dst_v, rows_v, acc_sh, sem):
    c = lax.axis_index("c")
    s = lax.axis_index("s")
    wid = c * NS + s
    pltpu.sync_copy(zeros_hbm, acc_sh.at[pl.ds(s * RPT, RPT)])

    @pl.when(s == 0)
    def _():
        pltpu.sync_copy(zeros_hbm.at[pl.ds(0, TAIL)],
                        acc_sh.at[pl.ds(NS * RPT, TAIL)])

    plsc.subcore_barrier()
    base = wid * EPW

    def body(ci, carry):
        off = pl.multiple_of(base + ci * CH, 8)
        pltpu.sync_copy(src_hbm.at[pl.ds(off, CH)], src_v)
        pltpu.sync_copy(dst_hbm.at[pl.ds(off, CH)], dst_v)
        pltpu.async_copy(hs_hbm.at[src_v], rows_v, sem).wait()
        pltpu.sync_copy(rows_v, acc_sh.at[dst_v], add=True)
        return carry

    lax.fori_loop(0, NCH, body, 0)
    plsc.subcore_barrier()
    pltpu.sync_copy(acc_sh.at[pl.ds(s * RPT, RPT)],
                    out_hbm.at[c, pl.ds(s * RPT, RPT)])

    @pl.when(s == 0)
    def _():
        pltpu.sync_copy(acc_sh.at[pl.ds(NS * RPT, TAIL)],
                        out_hbm.at[c, pl.ds(NS * RPT, TAIL)])


def _row_spec(w):
    return pl.BlockSpec((BLK, w), lambda i: (i, 0))


def _full_spec(shape):
    nd = len(shape)
    return pl.BlockSpec(shape, lambda i: (0,) * nd)


def _dinv(d0, d1):
    deg = 1.0 + d0[:, 0:1] + d1[:, 0:1]
    return lax.rsqrt(deg)


def _kan_body(x_ref, d0_ref, d1_ref, w1t_ref, b1_ref, w2t_ref, b2_ref, hs_ref):
    dinv = _dinv(d0_ref[...], d1_ref[...])
    h = jnp.dot(x_ref[...], w1t_ref[...], preferred_element_type=jnp.float32)
    h = jax.nn.gelu(h + b1_ref[...])
    h = jnp.dot(h, w2t_ref[...], preferred_element_type=jnp.float32) + b2_ref[...]
    hs_ref[...] = h * dinv


def _tc_kan_scale(x, d0, d1, w1t, b1, w2t, b2):
    return pl.pallas_call(
        _kan_body,
        grid=(NBLK,),
        in_specs=[_row_spec(D), _row_spec(DEGW), _row_spec(DEGW),
                  _full_spec((D, D)), _full_spec((1, D)),
                  _full_spec((D, D)), _full_spec((1, D))],
        out_specs=_row_spec(D),
        out_shape=jax.ShapeDtypeStruct((N, D), jnp.float32),
    )(x, d0, d1, w1t, b1, w2t, b2)


def _comb_body(a0_ref, a1_ref, hs_ref, d0_ref, d1_ref, b_ref, v_ref, st_ref):
    dinv = _dinv(d0_ref[...], d1_ref[...])
    v = dinv * (a0_ref[...] + a1_ref[...] + hs_ref[...]) + b_ref[...]
    v_ref[...] = v
    s1 = jnp.sum(v, axis=0, keepdims=True)
    s2 = jnp.sum(v * v, axis=0, keepdims=True)
    st = jnp.concatenate([s1, s2, jnp.zeros((6, D), v.dtype)], axis=0)

    @pl.when(pl.program_id(0) == 0)
    def _():
        st_ref[...] = st

    @pl.when(pl.program_id(0) > 0)
    def _():
        st_ref[...] = st_ref[...] + st


def _tc_combine(a0, a1, hs, d0, d1, bias):
    return pl.pallas_call(
        _comb_body,
        grid=(NBLK,),
        in_specs=[_row_spec(D), _row_spec(D), _row_spec(D),
                  _row_spec(DEGW), _row_spec(DEGW), _full_spec((1, D))],
        out_specs=[_row_spec(D), _full_spec((8, D))],
        out_shape=[jax.ShapeDtypeStruct((N, D), jnp.float32),
                   jax.ShapeDtypeStruct((8, D), jnp.float32)],
    )(a0, a1, hs, d0, d1, bias)


def _bn(v, st, gamma, beta):
    mu = st[0:1, :] * (1.0 / N)
    var = st[1:2, :] * (1.0 / N) - mu * mu
    return gamma * (v - mu) * lax.rsqrt(var + 1e-5) + beta


def _bnkan_body(v_ref, st_ref, g_ref, be_ref, w1t_ref, b1_ref, w2t_ref, b2_ref,
                d0_ref, d1_ref, h_ref, hs_ref):
    dinv = _dinv(d0_ref[...], d1_ref[...])
    h = _bn(v_ref[...], st_ref[...], g_ref[...], be_ref[...])
    h_ref[...] = h
    t = jnp.dot(h, w1t_ref[...], preferred_element_type=jnp.float32)
    t = jax.nn.gelu(t + b1_ref[...])
    t = jnp.dot(t, w2t_ref[...], preferred_element_type=jnp.float32) + b2_ref[...]
    hs_ref[...] = t * dinv


def _tc_bn_kan_scale(v, st, gamma, beta, w1t, b1, w2t, b2, d0, d1):
    return pl.pallas_call(
        _bnkan_body,
        grid=(NBLK,),
        in_specs=[_row_spec(D), _full_spec((8, D)),
                  _full_spec((1, D)), _full_spec((1, D)),
                  _full_spec((D, D)), _full_spec((1, D)),
                  _full_spec((D, D)), _full_spec((1, D)),
                  _row_spec(DEGW), _row_spec(DEGW)],
        out_specs=[_row_spec(D), _row_spec(D)],
        out_shape=[jax.ShapeDtypeStruct((N, D), jnp.float32),
                   jax.ShapeDtypeStruct((N, D), jnp.float32)],
    )(v, st, gamma, beta, w1t, b1, w2t, b2, d0, d1)


_G = [i * 0.5 - 2.5 for i in range(11)]


def _head_body(x_ref, h1_ref, v2_ref, st_ref, g_ref, be_ref, bwt_ref, swt_ref,
               out_ref):
    h2 = _bn(v2_ref[...], st_ref[...], g_ref[...], be_ref[...])
    z = jnp.concatenate([x_ref[...], h1_ref[...], h2], axis=1)
    out = jnp.dot(jax.nn.silu(z), bwt_ref[...],
                  preferred_element_type=jnp.float32)
    bas = [((z >= _G[i]) & (z < _G[i + 1])).astype(z.dtype) for i in range(10)]
    for p in range(1, 4):
        bas = [(z - _G[i]) / (_G[i + p] - _G[i]) * bas[i]
               + (_G[i + p + 1] - z) / (_G[i + p + 1] - _G[i + 1]) * bas[i + 1]
               for i in range(10 - p)]
    for j in range(7):
        out = out + jnp.dot(bas[j], swt_ref[j],
                            preferred_element_type=jnp.float32)
    out_ref[...] = out


def _tc_head(x, h1, v2, st, gamma, beta, bwt, swt):
    return pl.pallas_call(
        _head_body,
        grid=(NBLK,),
        in_specs=[_row_spec(D), _row_spec(D), _row_spec(D), _full_spec((8, D)),
                  _full_spec((1, D)), _full_spec((1, D)),
                  _full_spec((3 * D, C_OUT)), _full_spec((7, 3 * D, C_OUT))],
        out_specs=_row_spec(C_OUT),
        out_shape=jax.ShapeDtypeStruct((N, C_OUT), jnp.float32),
    )(x, h1, v2, st, gamma, beta, bwt, swt)


def kernel(x, edge_index, fc1_w0, fc1_b0, fc2_w0, fc2_b0, conv_b0, gamma0,
           beta0, fc1_w1, fc1_b1, fc2_w1, fc2_b1, conv_b1, gamma1, beta1,
           base_w, spline_w):
    src = edge_index[0]
    dst = edge_index[1]
    zeros_d = jnp.zeros((RPT, D), jnp.float32)
    zeros_w = jnp.zeros((RPT, DEGW), jnp.float32)
    ones_w = jnp.ones((CH, DEGW), jnp.float32)

    degp = _sc_degree(dst, ones_w, zeros_w)
    d0, d1 = degp[0], degp[1]

    hs0 = _tc_kan_scale(x, d0, d1, fc1_w0.T, fc1_b0[None], fc2_w0.T,
                        fc2_b0[None])
    acc0 = _sc_aggregate(src, dst, hs0, zeros_d)
    v1, st1 = _tc_combine(acc0[0], acc0[1], hs0, d0, d1, conv_b0[None])
    h1, hs1 = _tc_bn_kan_scale(v1, st1, gamma0[None], beta0[None], fc1_w1.T,
                               fc1_b1[None], fc2_w1.T, fc2_b1[None], d0, d1)

    acc1 = _sc_aggregate(src, dst, hs1, zeros_d)
    v2, st2 = _tc_combine(acc1[0], acc1[1], hs1, d0, d1, conv_b1[None])

    swt = jnp.transpose(spline_w, (2, 1, 0))
    return _tc_head(x, h1, v2, st2, gamma1[None], beta1[None], base_w.T, swt)


# speedup vs baseline: 17.1596x; 1.6844x over previous
import functools

import jax
import jax.numpy as jnp
from jax import lax
from jax.experimental import pallas as pl
from jax.experimental.pallas import tpu as pltpu
from jax.experimental.pallas import tpu_sc as plsc

N = 10000
E = 320000
D = 128
C_OUT = 40

NC = 2
NS = 16
NW = NC * NS
EPW = E // NW
CH = 80
NCH = EPW // CH
RPT = 624
TAIL = N - NS * RPT
DEGW = 128

BLK = 1000
NBLK = N // BLK

_sc_mesh = plsc.VectorSubcoreMesh(core_axis_name="c", subcore_axis_name="s")


NGRP = NCH // 2
TAILC = NCH - 1


@functools.partial(
    pl.kernel,
    mesh=_sc_mesh,
    out_type=jax.ShapeDtypeStruct((NC, N, D), jnp.float32),
    scratch_types=[
        pltpu.VMEM((NCH, CH), jnp.int32),
        pltpu.VMEM((CH,), jnp.int32),
        pltpu.VMEM((CH,), jnp.int32),
        pltpu.VMEM((2, CH, D), jnp.float32),
        pltpu.VMEM_SHARED((N, D), jnp.float32),
        pltpu.SemaphoreType.DMA((2,)),
        pltpu.SemaphoreType.DMA((2,)),
    ],
)
def _sc_aggregate(src_hbm, dst_hbm, hs_hbm, zeros_hbm, out_hbm,
                  src_v, dst0_v, dst1_v, rows_v, acc_sh, gsems, dsems):
    c = lax.axis_index("c")
    s = lax.axis_index("s")
    wid = c * NS + s
    pltpu.sync_copy(zeros_hbm, acc_sh.at[pl.ds(s * RPT, RPT)])

    @pl.when(s == 0)
    def _():
        pltpu.sync_copy(zeros_hbm.at[pl.ds(0, TAIL)],
                        acc_sh.at[pl.ds(NS * RPT, TAIL)])

    pltpu.sync_copy(src_hbm.at[wid], src_v)
    plsc.subcore_barrier()

    dbufs = (dst0_v, dst1_v)
    pltpu.async_copy(dst_hbm.at[wid, 0], dst0_v, dsems.at[0])
    pltpu.async_copy(dst_hbm.at[wid, 1], dst1_v, dsems.at[1])
    pltpu.async_copy(hs_hbm.at[src_v.at[0]], rows_v.at[0], gsems.at[0])
    pltpu.async_copy(hs_hbm.at[src_v.at[1]], rows_v.at[1], gsems.at[1])

    def visit(ci, j):
        pltpu.make_async_copy(hs_hbm.at[src_v.at[ci]], rows_v.at[j],
                              gsems.at[j]).wait()
        pltpu.make_async_copy(dst_hbm.at[wid, 0], dbufs[j],
                              dsems.at[j]).wait()
        pltpu.sync_copy(rows_v.at[j], acc_sh.at[dbufs[j]], add=True)

        @pl.when(ci + 2 < NCH)
        def _():
            pltpu.async_copy(dst_hbm.at[wid, ci + 2], dbufs[j], dsems.at[j])
            pltpu.async_copy(hs_hbm.at[src_v.at[ci + 2]], rows_v.at[j],
                             gsems.at[j])

    def body(g, carry):
        visit(2 * g, 0)
        visit(2 * g + 1, 1)
        return carry

    lax.fori_loop(0, NGRP, body, 0)
    visit(TAILC, TAILC % 2)
    plsc.subcore_barrier()
    pltpu.sync_copy(acc_sh.at[pl.ds(s * RPT, RPT)],
                    out_hbm.at[c, pl.ds(s * RPT, RPT)])

    @pl.when(s == 0)
    def _():
        pltpu.sync_copy(acc_sh.at[pl.ds(NS * RPT, TAIL)],
                        out_hbm.at[c, pl.ds(NS * RPT, TAIL)])


def _row_spec(w):
    return pl.BlockSpec((BLK, w), lambda i: (i, 0))


def _full_spec(shape):
    nd = len(shape)
    return pl.BlockSpec(shape, lambda i: (0,) * nd)


def _dinv(d0, d1):
    deg = 1.0 + d0[:, 0:1] + d1[:, 0:1]
    return lax.rsqrt(deg)


def _kan_body(x_ref, d0_ref, d1_ref, w1t_ref, b1_ref, w2t_ref, b2_ref, hs_ref):
    dinv = _dinv(d0_ref[...], d1_ref[...])
    h = jnp.dot(x_ref[...], w1t_ref[...], preferred_element_type=jnp.float32)
    h = jax.nn.gelu(h + b1_ref[...])
    h = jnp.dot(h, w2t_ref[...], preferred_element_type=jnp.float32) + b2_ref[...]
    hs_ref[...] = h * dinv


def _tc_kan_scale(x, d0, d1, w1t, b1, w2t, b2):
    return pl.pallas_call(
        _kan_body,
        grid=(NBLK,),
        in_specs=[_row_spec(D), _row_spec(DEGW), _row_spec(DEGW),
                  _full_spec((D, D)), _full_spec((1, D)),
                  _full_spec((D, D)), _full_spec((1, D))],
        out_specs=_row_spec(D),
        out_shape=jax.ShapeDtypeStruct((N, D), jnp.float32),
    )(x, d0, d1, w1t, b1, w2t, b2)


def _comb_body(a0_ref, a1_ref, hs_ref, d0_ref, d1_ref, b_ref, v_ref, st_ref):
    dinv = _dinv(d0_ref[...], d1_ref[...])
    v = dinv * (a0_ref[...] + a1_ref[...] + hs_ref[...]) + b_ref[...]
    v_ref[...] = v
    s1 = jnp.sum(v, axis=0, keepdims=True)
    s2 = jnp.sum(v * v, axis=0, keepdims=True)
    st = jnp.concatenate([s1, s2, jnp.zeros((6, D), v.dtype)], axis=0)

    @pl.when(pl.program_id(0) == 0)
    def _():
        st_ref[...] = st

    @pl.when(pl.program_id(0) > 0)
    def _():
        st_ref[...] = st_ref[...] + st


def _tc_combine(a0, a1, hs, d0, d1, bias):
    return pl.pallas_call(
        _comb_body,
        grid=(NBLK,),
        in_specs=[_row_spec(D), _row_spec(D), _row_spec(D),
                  _row_spec(DEGW), _row_spec(DEGW), _full_spec((1, D))],
        out_specs=[_row_spec(D), _full_spec((8, D))],
        out_shape=[jax.ShapeDtypeStruct((N, D), jnp.float32),
                   jax.ShapeDtypeStruct((8, D), jnp.float32)],
    )(a0, a1, hs, d0, d1, bias)


def _bn(v, st, gamma, beta):
    mu = st[0:1, :] * (1.0 / N)
    var = st[1:2, :] * (1.0 / N) - mu * mu
    return gamma * (v - mu) * lax.rsqrt(var + 1e-5) + beta


def _bnkan_body(v_ref, st_ref, g_ref, be_ref, w1t_ref, b1_ref, w2t_ref, b2_ref,
                d0_ref, d1_ref, h_ref, hs_ref):
    dinv = _dinv(d0_ref[...], d1_ref[...])
    h = _bn(v_ref[...], st_ref[...], g_ref[...], be_ref[...])
    h_ref[...] = h
    t = jnp.dot(h, w1t_ref[...], preferred_element_type=jnp.float32)
    t = jax.nn.gelu(t + b1_ref[...])
    t = jnp.dot(t, w2t_ref[...], preferred_element_type=jnp.float32) + b2_ref[...]
    hs_ref[...] = t * dinv


def _tc_bn_kan_scale(v, st, gamma, beta, w1t, b1, w2t, b2, d0, d1):
    return pl.pallas_call(
        _bnkan_body,
        grid=(NBLK,),
        in_specs=[_row_spec(D), _full_spec((8, D)),
                  _full_spec((1, D)), _full_spec((1, D)),
                  _full_spec((D, D)), _full_spec((1, D)),
                  _full_spec((D, D)), _full_spec((1, D)),
                  _row_spec(DEGW), _row_spec(DEGW)],
        out_specs=[_row_spec(D), _row_spec(D)],
        out_shape=[jax.ShapeDtypeStruct((N, D), jnp.float32),
                   jax.ShapeDtypeStruct((N, D), jnp.float32)],
    )(v, st, gamma, beta, w1t, b1, w2t, b2, d0, d1)


_G = [i * 0.5 - 2.5 for i in range(11)]


def _head_body(x_ref, h1_ref, v2_ref, st_ref, g_ref, be_ref, bwt_ref, swt_ref,
               out_ref):
    h2 = _bn(v2_ref[...], st_ref[...], g_ref[...], be_ref[...])
    z = jnp.concatenate([x_ref[...], h1_ref[...], h2], axis=1)
    out = jnp.dot(jax.nn.silu(z), bwt_ref[...],
                  preferred_element_type=jnp.float32)
    bas = [((z >= _G[i]) & (z < _G[i + 1])).astype(z.dtype) for i in range(10)]
    for p in range(1, 4):
        bas = [(z - _G[i]) / (_G[i + p] - _G[i]) * bas[i]
               + (_G[i + p + 1] - z) / (_G[i + p + 1] - _G[i + 1]) * bas[i + 1]
               for i in range(10 - p)]
    for j in range(7):
        out = out + jnp.dot(bas[j], swt_ref[j],
                            preferred_element_type=jnp.float32)
    out_ref[...] = out


def _tc_head(x, h1, v2, st, gamma, beta, bwt, swt):
    return pl.pallas_call(
        _head_body,
        grid=(NBLK,),
        in_specs=[_row_spec(D), _row_spec(D), _row_spec(D), _full_spec((8, D)),
                  _full_spec((1, D)), _full_spec((1, D)),
                  _full_spec((3 * D, C_OUT)), _full_spec((7, 3 * D, C_OUT))],
        out_specs=_row_spec(C_OUT),
        out_shape=jax.ShapeDtypeStruct((N, C_OUT), jnp.float32),
    )(x, h1, v2, st, gamma, beta, bwt, swt)


def kernel(x, edge_index, fc1_w0, fc1_b0, fc2_w0, fc2_b0, conv_b0, gamma0,
           beta0, fc1_w1, fc1_b1, fc2_w1, fc2_b1, conv_b1, gamma1, beta1,
           base_w, spline_w):
    src = edge_index[0].reshape(NW, NCH, CH)
    dst = edge_index[1].reshape(NW, NCH, CH)
    zeros_d = jnp.zeros((RPT, D), jnp.float32)
    ones_nd = jnp.ones((N, D), jnp.float32)

    degp = _sc_aggregate(src, dst, ones_nd, zeros_d)
    d0, d1 = degp[0], degp[1]

    hs0 = _tc_kan_scale(x, d0, d1, fc1_w0.T, fc1_b0[None], fc2_w0.T,
                        fc2_b0[None])
    acc0 = _sc_aggregate(src, dst, hs0, zeros_d)
    v1, st1 = _tc_combine(acc0[0], acc0[1], hs0, d0, d1, conv_b0[None])
    h1, hs1 = _tc_bn_kan_scale(v1, st1, gamma0[None], beta0[None], fc1_w1.T,
                               fc1_b1[None], fc2_w1.T, fc2_b1[None], d0, d1)

    acc1 = _sc_aggregate(src, dst, hs1, zeros_d)
    v2, st2 = _tc_combine(acc1[0], acc1[1], hs1, d0, d1, conv_b1[None])

    swt = jnp.transpose(spline_w, (2, 1, 0))
    return _tc_head(x, h1, v2, st2, gamma1[None], beta1[None], base_w.T, swt)
